# Initial kernel scaffold; baseline (speedup 1.0000x reference)
import jax
import jax.numpy as jnp
from jax.experimental import pallas as pl


def kernel(x, batch_indices):
    raise NotImplementedError("write your pallas kernel here")


# speedup vs baseline: 9.0271x; 9.0271x over previous
import jax
import jax.numpy as jnp
from jax import lax
from jax.experimental import pallas as pl
from jax.experimental.pallas import tpu as pltpu

_N = 100000
_D = 128
_S = 256
_BLOCK = 2000
_NBLK = _N // _BLOCK


def _pool_kernel(idx_ref, x_ref, out_ref, acc_ref, cnt_ref):
    i = pl.program_id(0)

    idx = idx_ref[0, 0, :]
    xb = x_ref[...].astype(jnp.bfloat16)
    seg = lax.broadcasted_iota(jnp.int32, (_S, _BLOCK), 0)
    oh = (seg == idx[None, :]).astype(jnp.bfloat16)
    psum = lax.dot(oh, xb, preferred_element_type=jnp.float32)
    ones = jnp.ones((_BLOCK, _D), jnp.bfloat16)
    pcnt = lax.dot(oh, ones, preferred_element_type=jnp.float32)

    @pl.when(i == 0)
    def _init():
        acc_ref[...] = psum
        cnt_ref[...] = pcnt

    @pl.when(i > 0)
    def _acc():
        acc_ref[...] += psum
        cnt_ref[...] += pcnt

    @pl.when(i == _NBLK - 1)
    def _fin():
        out_ref[...] = acc_ref[...] / jnp.maximum(cnt_ref[...], 1.0)


def kernel(x, batch_indices):
    idx3 = batch_indices.astype(jnp.int32).reshape(_NBLK, 1, _BLOCK)
    out = pl.pallas_call(
        _pool_kernel,
        grid=(_NBLK,),
        in_specs=[
            pl.BlockSpec((1, 1, _BLOCK), lambda i: (i, 0, 0)),
            pl.BlockSpec((_BLOCK, _D), lambda i: (i, 0)),
        ],
        out_specs=pl.BlockSpec((_S, _D), lambda i: (0, 0)),
        out_shape=jax.ShapeDtypeStruct((_S, _D), jnp.float32),
        scratch_shapes=[
            pltpu.VMEM((_S, _D), jnp.float32),
            pltpu.VMEM((_S, _D), jnp.float32),
        ],
    )(idx3, x)
    return (out, None)
